# baseline (speedup 1.0000x reference)
import functools

import jax
import jax.numpy as jnp
from jax import lax
from jax.experimental import pallas as pl
from jax.experimental.pallas import tpu as pltpu
from jax.experimental.pallas import tpu_sc as plsc

_SIZE = 65536
_DIM = 256
_ROWS = 2048
_QB = 1024
_NBLK = _SIZE // _QB
_EPS = 1e-12


_G = _NBLK // 2


def _argmax_compute(tref, b, colf):
    t = tref[...]
    m = jnp.max(t, axis=1, keepdims=True)
    col = jnp.broadcast_to(colf[...], t.shape)
    arg = (jnp.min(jnp.where(t == m, col, jnp.float32(_QB)), axis=1,
                   keepdims=True)
           + (b * _QB).astype(jnp.float32))
    return m, arg


def _best_update(b, m, arg, bestv, besti):
    bv = bestv[...]
    valid = jnp.logical_and(b >= 0, b < _NBLK)
    upd = jnp.logical_and(valid, jnp.logical_or(m > bv, b == 0))
    bestv[...] = jnp.where(upd, m, bv)
    besti[...] = jnp.where(upd, arg, besti[...])


def _matmul_block(nx, q_ref):
    q = q_ref[...]
    qnorm = jnp.maximum(jnp.sqrt(jnp.sum(q * q, axis=1, keepdims=True)), _EPS)
    qn = q / qnorm
    return lax.dot_general(nx[...], qn, (((1,), (1,)), ((), ())),
                           preferred_element_type=jnp.float32)


def _topk_body(x_ref, qa_ref, qb_ref, idx_out, sim_out,
               bestv, besti, nx, colf, ta, tb):
    pid = pl.program_id(0)

    @pl.when(pid == 0)
    def _():
        xv = x_ref[...]
        xnorm = jnp.maximum(jnp.sqrt(jnp.sum(xv * xv, axis=1, keepdims=True)), _EPS)
        nx[...] = xv / xnorm
        colf[...] = lax.broadcasted_iota(jnp.int32, (1, _QB), 1).astype(jnp.float32)

    m1, a1 = _argmax_compute(tb, 2 * pid - 1, colf)
    ta[...] = _matmul_block(nx, qa_ref)
    _best_update(2 * pid - 1, m1, a1, bestv, besti)
    m2, a2 = _argmax_compute(ta, 2 * pid, colf)
    tb[...] = _matmul_block(nx, qb_ref)
    _best_update(2 * pid, m2, a2, bestv, besti)

    @pl.when(pid == _G)
    def _():
        idx_out[...] = besti[...].astype(jnp.int32)
        sim_out[0, 0] = jnp.sum(bestv[...]) / _ROWS


_topk = pl.pallas_call(
    _topk_body,
    grid=(_G + 1,),
    in_specs=[
        pl.BlockSpec((_ROWS, _DIM), lambda i: (0, 0)),
        pl.BlockSpec((_QB, _DIM),
                     lambda i: (jnp.minimum(2 * i, _NBLK - 1), 0)),
        pl.BlockSpec((_QB, _DIM),
                     lambda i: (jnp.minimum(2 * i + 1, _NBLK - 1), 0)),
    ],
    out_specs=[
        pl.BlockSpec((_ROWS, 1), lambda i: (0, 0)),
        pl.BlockSpec(memory_space=pltpu.SMEM),
    ],
    out_shape=[
        jax.ShapeDtypeStruct((_ROWS, 1), jnp.int32),
        jax.ShapeDtypeStruct((1, 1), jnp.float32),
    ],
    scratch_shapes=[pltpu.VMEM((_ROWS, 1), jnp.float32),
                    pltpu.VMEM((_ROWS, 1), jnp.float32),
                    pltpu.VMEM((_ROWS, _DIM), jnp.float32),
                    pltpu.VMEM((1, _QB), jnp.float32),
                    pltpu.VMEM((_ROWS, _QB), jnp.float32),
                    pltpu.VMEM((_ROWS, _QB), jnp.float32)],
)


_NC, _NS, _L = 2, 16, 16
_NW = _NC * _NS
_BPW = _ROWS // _NW


@functools.partial(
    pl.kernel,
    mesh=plsc.VectorSubcoreMesh(core_axis_name="c", subcore_axis_name="s"),
    out_type=[
        jax.ShapeDtypeStruct((_ROWS, _DIM), jnp.float32),
        jax.ShapeDtypeStruct((_ROWS,), jnp.int32),
    ],
    scratch_types=[
        pltpu.VMEM((_BPW,), jnp.int32),
        pltpu.VMEM((_BPW, _DIM), jnp.float32),
        pltpu.VMEM((_BPW,), jnp.int32),
        pltpu.SemaphoreType.DMA,
        pltpu.SemaphoreType.DMA,
    ],
)
def _gather(table_hbm, idx_hbm, age_hbm, rows_out, age_out,
            idx_v, rows_v, ageo_v, sem, sem2):
    wid = lax.axis_index("s") * _NC + lax.axis_index("c")
    base = wid * _BPW
    pltpu.sync_copy(idx_hbm.at[pl.ds(base, _BPW)], idx_v)
    cp1 = pltpu.async_copy(table_hbm.at[idx_v], rows_v, sem)
    cp2 = pltpu.async_copy(age_hbm.at[idx_v], ageo_v, sem2)
    cp1.wait()
    cp2.wait()
    pltpu.sync_copy(rows_v, rows_out.at[pl.ds(base, _BPW)])
    pltpu.sync_copy(ageo_v, age_out.at[pl.ds(base, _BPW)])


def kernel(x, idx, queue_x, age):
    del idx
    best2, simmean = _topk(x, queue_x, queue_x)
    best_idx = best2.reshape(_ROWS)
    nn_x, age_g = _gather(queue_x, best_idx, age)
    nn_similarity = simmean[0, 0]
    nn_age = jnp.mean(age_g.astype(jnp.float32))
    return nn_x, nn_similarity, nn_age
